# baseline (speedup 1.0000x reference)
import functools

import jax
import jax.numpy as jnp
from jax import lax
from jax.experimental import pallas as pl
from jax.experimental.pallas import tpu as pltpu
from jax.experimental.pallas import tpu_sc as plsc

_TOKENS = 32
_VOCAB = 1_000_000
_EPS = 1e-10

_NC = 2
_NS = 16
_L = 16

_G = 4
_TPG = 8
_P = 8
_CT = 16
_CHUNK = _CT * 128
_NCH = 32
_SHARD = _NCH * _CHUNK
_TC_OFF = _P * _SHARD
_TC_BW = _SHARD
_TC_NB = -(-(_VOCAB - _TC_OFF) // _TC_BW)
_BIG = 2**31 - 1


def _noise_table_body(exp_ref, out_ref):
    out_ref[...] = jnp.log(exp_ref[0, :] + _EPS)


def _make_noise_table(exponential):
    return pl.pallas_call(
        _noise_table_body,
        grid=(_P,),
        in_specs=[pl.BlockSpec((1, _SHARD), lambda i: (0, i))],
        out_specs=pl.BlockSpec((_SHARD,), lambda i: (i,)),
        out_shape=jax.ShapeDtypeStruct((_TC_OFF,), jnp.float32),
    )(exponential)


def _chunk_scan(lbuf, nbuf, invts, state, ch):
    def body(c, carry):
        st = list(carry)
        nv = [nbuf[pl.ds(c * 128 + 16 * k, _L)] for k in range(8)]
        tsplat = jnp.full((_L,), ch * _CT + c, jnp.int32)
        for r in range(_TPG):
            t = lbuf[r, pl.ds(c * 128, _L)] * invts[r] - nv[0]
            for k in range(1, 8):
                t = jnp.maximum(
                    t, lbuf[r, pl.ds(c * 128 + 16 * k, _L)] * invts[r] - nv[k])
            upd = t > st[2 * r]
            st[2 * r] = jnp.where(upd, t, st[2 * r])
            st[2 * r + 1] = jnp.where(upd, tsplat, st[2 * r + 1])
        return tuple(st)

    return list(lax.fori_loop(0, _CT, body, tuple(state)))


def _locate(ltile, nbuf, nsub, invt_r, r, base_idx, iota):
    m = jnp.full((_L,), -jnp.inf, jnp.float32)
    ix = jnp.zeros((_L,), jnp.int32)
    iv0 = iota + base_idx
    for k in range(8):
        s = (ltile[r, pl.ds(16 * k, _L)] * invt_r
             - nbuf[pl.ds(nsub + 16 * k, _L)])
        upd = s > m
        m = jnp.where(upd, s, m)
        ix = jnp.where(upd, iv0 + 16 * k, ix)
    return m, ix


def _sampler_body(logits, invt, ntab, pval, pidx,
                  lbufA, lbufB, nbufA, nbufB,
                  invt_v, sval, sidx,
                  semLA, semLB, semNA, semNB):
    wid = lax.axis_index("s") * _NC + lax.axis_index("c")
    g = lax.rem(wid, _G)
    p = lax.div(wid, _G)
    rows0 = pl.multiple_of(g * _TPG, _TPG)
    base = p * _SHARD
    iota = lax.iota(jnp.int32, _L)

    pltpu.sync_copy(invt, invt_v)
    invts = [
        plsc.load_gather(invt_v, [jnp.full((_L,), rows0 + r, jnp.int32)])
        for r in range(_TPG)
    ]

    def start(ch, lbuf, nbuf, semL, semN):
        off = pl.multiple_of(base + ch * _CHUNK, 1024)
        pltpu.async_copy(
            logits.at[pl.ds(rows0, _TPG), pl.ds(off, _CHUNK)], lbuf, semL)
        pltpu.async_copy(ntab.at[pl.ds(off, _CHUNK)], nbuf, semN)

    def wait(lbuf, nbuf, semL, semN):
        pltpu.make_async_copy(
            logits.at[pl.ds(0, _TPG), pl.ds(0, _CHUNK)], lbuf, semL).wait()
        pltpu.make_async_copy(ntab.at[pl.ds(0, _CHUNK)], nbuf, semN).wait()

    start(0, lbufA, nbufA, semLA, semNA)
    start(1, lbufB, nbufB, semLB, semNB)

    state0 = ()
    for _ in range(_TPG):
        state0 += (jnp.full((_L,), -jnp.inf, jnp.float32),
                   jnp.zeros((_L,), jnp.int32))

    def pair(i, carry):
        st = list(carry)
        c0 = 2 * i
        wait(lbufA, nbufA, semLA, semNA)
        st = _chunk_scan(lbufA, nbufA, invts, st, c0)

        @pl.when(c0 + 2 < _NCH)
        def _():
            start(c0 + 2, lbufA, nbufA, semLA, semNA)

        wait(lbufB, nbufB, semLB, semNB)
        st = _chunk_scan(lbufB, nbufB, invts, st, c0 + 1)

        @pl.when(c0 + 3 < _NCH)
        def _():
            start(c0 + 3, lbufB, nbufB, semLB, semNB)

        return tuple(st)

    st = list(lax.fori_loop(0, _NCH // 2, pair, state0))

    tids = []
    for r in range(_TPG):
        best = jnp.max(st[2 * r])
        tids.append(jnp.min(jnp.where(st[2 * r] == best, st[2 * r + 1], _BIG)))

    def start2(tid, lbuf, nbuf, semL, semN):
        loff = pl.multiple_of(base + tid * 128, 128)
        noff = pl.multiple_of(base + lax.div(tid, 8) * 1024, 1024)
        pltpu.async_copy(
            logits.at[pl.ds(rows0, _TPG), pl.ds(loff, 128)],
            lbuf.at[:, pl.ds(0, 128)], semL)
        pltpu.async_copy(ntab.at[pl.ds(noff, 1024)],
                         nbuf.at[pl.ds(0, 1024)], semN)

    def wait2(lbuf, nbuf, semL, semN):
        pltpu.make_async_copy(
            logits.at[pl.ds(0, _TPG), pl.ds(0, 128)],
            lbuf.at[:, pl.ds(0, 128)], semL).wait()
        pltpu.make_async_copy(
            ntab.at[pl.ds(0, 1024)], nbuf.at[pl.ds(0, 1024)], semN).wait()

    start2(tids[0], lbufA, nbufA, semLA, semNA)
    start2(tids[1], lbufB, nbufB, semLB, semNB)
    for r in range(_TPG):
        lbuf, nbuf = (lbufA, nbufA) if r % 2 == 0 else (lbufB, nbufB)
        semL, semN = (semLA, semNA) if r % 2 == 0 else (semLB, semNB)
        wait2(lbuf, nbuf, semL, semN)
        nsub = lax.rem(tids[r], 8) * 128
        m, ix = _locate(lbuf, nbuf, nsub, invts[r], r,
                        base + tids[r] * 128, iota)
        if r + 2 < _TPG:
            start2(tids[r + 2], lbuf, nbuf, semL, semN)
        sval[r, :] = m
        sidx[r, :] = ix
    pltpu.sync_copy(sval, pval.at[wid])
    pltpu.sync_copy(sidx, pidx.at[wid])


_sampler = functools.partial(
    pl.kernel,
    out_type=(
        jax.ShapeDtypeStruct((_NC * _NS, _TPG, _L), jnp.float32),
        jax.ShapeDtypeStruct((_NC * _NS, _TPG, _L), jnp.int32),
    ),
    mesh=plsc.VectorSubcoreMesh(
        core_axis_name="c", subcore_axis_name="s",
        num_cores=_NC, num_subcores=_NS),
    compiler_params=pltpu.CompilerParams(needs_layout_passes=False),
    scratch_types=[
        pltpu.VMEM((_TPG, _CHUNK), jnp.float32),
        pltpu.VMEM((_TPG, _CHUNK), jnp.float32),
        pltpu.VMEM((_CHUNK,), jnp.float32),
        pltpu.VMEM((_CHUNK,), jnp.float32),
        pltpu.VMEM((_TOKENS,), jnp.float32),
        pltpu.VMEM((_TPG, _L), jnp.float32),
        pltpu.VMEM((_TPG, _L), jnp.int32),
        pltpu.SemaphoreType.DMA,
        pltpu.SemaphoreType.DMA,
        pltpu.SemaphoreType.DMA,
        pltpu.SemaphoreType.DMA,
    ],
)(_sampler_body)


def _suffix_body(l_ref, e_ref, invt_ref, ov_ref, oi_ref):
    b = pl.program_id(0)
    base = _TC_OFF + b * _TC_BW
    lane = jax.lax.broadcasted_iota(jnp.int32, (_TOKENS, _TC_BW), 1)
    nl = jnp.log(e_ref[0, :] + _EPS)
    s = l_ref[...] * invt_ref[...][:, None] - nl[None, :]
    s = jnp.where(base + lane < _VOCAB, s, -jnp.inf)
    ov_ref[...] = jnp.max(s, axis=1)[None, None, :]
    ai = jnp.argmax(s, axis=1).astype(jnp.int32) + base
    oi_ref[...] = ai[None, None, :]


def _suffix_scan(logits, exponential, invt):
    return pl.pallas_call(
        _suffix_body,
        grid=(_TC_NB,),
        in_specs=[
            pl.BlockSpec((_TOKENS, _TC_BW), lambda b: (0, _P + b)),
            pl.BlockSpec((1, _TC_BW), lambda b: (0, _P + b)),
            pl.BlockSpec((_TOKENS,), lambda b: (0,)),
        ],
        out_specs=[
            pl.BlockSpec((1, 1, _TOKENS), lambda b: (b, 0, 0)),
            pl.BlockSpec((1, 1, _TOKENS), lambda b: (b, 0, 0)),
        ],
        out_shape=[
            jax.ShapeDtypeStruct((_TC_NB, 1, _TOKENS), jnp.float32),
            jax.ShapeDtypeStruct((_TC_NB, 1, _TOKENS), jnp.int32),
        ],
    )(logits, exponential, invt)


def _merge_body(pv_ref, pi_ref, tv_ref, ti_ref, out_ref):
    v = pv_ref[...].reshape(_P, _G, _TPG, _L)
    ix = pi_ref[...].reshape(_P, _G, _TPG, _L)
    best = jnp.max(v, axis=(0, 3))
    tie = v == best[None, :, :, None]
    cand = jnp.where(tie, ix, _BIG)
    m = best.reshape(_TOKENS)
    i = jnp.min(cand, axis=(0, 3)).reshape(_TOKENS)

    tv = tv_ref[...]
    tmax = jnp.max(tv, axis=0)
    tcand = jnp.where(tv == tmax[None, :], ti_ref[...], _BIG)
    tidx = jnp.min(tcand, axis=0)
    out_ref[...] = jnp.where(tmax > m, tidx, i).astype(jnp.int32)


def _merge(pv, pi, tv, ti):
    nw = _NC * _NS
    return pl.pallas_call(
        _merge_body,
        grid=(1,),
        in_specs=[
            pl.BlockSpec((nw, _TPG, _L), lambda i: (0, 0, 0)),
            pl.BlockSpec((nw, _TPG, _L), lambda i: (0, 0, 0)),
            pl.BlockSpec((_TC_NB, _TOKENS), lambda i: (0, 0)),
            pl.BlockSpec((_TC_NB, _TOKENS), lambda i: (0, 0)),
        ],
        out_specs=pl.BlockSpec((_TOKENS,), lambda i: (0,)),
        out_shape=jax.ShapeDtypeStruct((_TOKENS,), jnp.int32),
    )(pv, pi, tv, ti)


@jax.jit
def kernel(logits, temperatures, exponential):
    ntab = _make_noise_table(exponential)
    pos = temperatures > 0
    invt = jnp.where(pos, 1.0 / jnp.where(pos, temperatures, 1.0), 1e30)
    pv, pi = _sampler(logits, invt, ntab)
    tv, ti = _suffix_scan(logits, exponential, invt)
    return _merge(pv, pi, tv.reshape(_TC_NB, _TOKENS),
                  ti.reshape(_TC_NB, _TOKENS))


# speedup vs baseline: 1.0301x; 1.0301x over previous
import functools

import jax
import jax.numpy as jnp
from jax import lax
from jax.experimental import pallas as pl
from jax.experimental.pallas import tpu as pltpu
from jax.experimental.pallas import tpu_sc as plsc

_TOKENS = 32
_VOCAB = 1_000_000
_EPS = 1e-10

_NC = 2
_NS = 16
_L = 16

_G = 4
_TPG = 8
_P = 8
_CT = 16
_CHUNK = _CT * 128
_NCH = 32
_SHARD = _NCH * _CHUNK
_TC_OFF = _P * _SHARD
_TC_BW = _SHARD
_TC_NB = -(-(_VOCAB - _TC_OFF) // _TC_BW)
_BIG = 2**31 - 1


def _noise_table_body(exp_ref, out_ref):
    out_ref[...] = jnp.log(exp_ref[0, :] + _EPS)


def _make_noise_table(exponential):
    blk = 131072 if _TC_OFF % 131072 == 0 else _SHARD
    return pl.pallas_call(
        _noise_table_body,
        grid=(_TC_OFF // blk,),
        in_specs=[pl.BlockSpec((1, blk), lambda i: (0, i))],
        out_specs=pl.BlockSpec((blk,), lambda i: (i,)),
        out_shape=jax.ShapeDtypeStruct((_TC_OFF,), jnp.float32),
    )(exponential)


def _chunk_scan(lbuf, nbuf, invts, state, ch):
    def body(c, carry):
        st = list(carry)
        nv = [nbuf[pl.ds(c * 128 + 16 * k, _L)] for k in range(8)]
        tsplat = jnp.full((_L,), ch * _CT + c, jnp.int32)
        for r in range(_TPG):
            t = lbuf[r, pl.ds(c * 128, _L)] * invts[r] - nv[0]
            for k in range(1, 8):
                t = jnp.maximum(
                    t, lbuf[r, pl.ds(c * 128 + 16 * k, _L)] * invts[r] - nv[k])
            upd = t > st[2 * r]
            st[2 * r] = jnp.where(upd, t, st[2 * r])
            st[2 * r + 1] = jnp.where(upd, tsplat, st[2 * r + 1])
        return tuple(st)

    return list(lax.fori_loop(0, _CT, body, tuple(state)))


def _locate(ltile, nbuf, nsub, invt_r, r, base_idx, iota):
    m = jnp.full((_L,), -jnp.inf, jnp.float32)
    ix = jnp.zeros((_L,), jnp.int32)
    iv0 = iota + base_idx
    for k in range(8):
        s = (ltile[r, pl.ds(16 * k, _L)] * invt_r
             - nbuf[pl.ds(nsub + 16 * k, _L)])
        upd = s > m
        m = jnp.where(upd, s, m)
        ix = jnp.where(upd, iv0 + 16 * k, ix)
    return m, ix


def _sampler_body(logits, invt, ntab, pval, pidx,
                  lbufA, lbufB, nbufA, nbufB,
                  invt_v, sval, sidx,
                  semLA, semLB, semNA, semNB):
    wid = lax.axis_index("s") * _NC + lax.axis_index("c")
    g = lax.rem(wid, _G)
    p = lax.div(wid, _G)
    rows0 = pl.multiple_of(g * _TPG, _TPG)
    base = p * _SHARD
    iota = lax.iota(jnp.int32, _L)

    pltpu.sync_copy(invt, invt_v)
    invts = [
        plsc.load_gather(invt_v, [jnp.full((_L,), rows0 + r, jnp.int32)])
        for r in range(_TPG)
    ]

    def start(ch, lbuf, nbuf, semL, semN):
        off = pl.multiple_of(base + ch * _CHUNK, 1024)
        pltpu.async_copy(
            logits.at[pl.ds(rows0, _TPG), pl.ds(off, _CHUNK)], lbuf, semL)
        pltpu.async_copy(ntab.at[pl.ds(off, _CHUNK)], nbuf, semN)

    def wait(lbuf, nbuf, semL, semN):
        pltpu.make_async_copy(
            logits.at[pl.ds(0, _TPG), pl.ds(0, _CHUNK)], lbuf, semL).wait()
        pltpu.make_async_copy(ntab.at[pl.ds(0, _CHUNK)], nbuf, semN).wait()

    start(0, lbufA, nbufA, semLA, semNA)
    start(1, lbufB, nbufB, semLB, semNB)

    state0 = ()
    for _ in range(_TPG):
        state0 += (jnp.full((_L,), -jnp.inf, jnp.float32),
                   jnp.zeros((_L,), jnp.int32))

    def pair(i, carry):
        st = list(carry)
        c0 = 2 * i
        wait(lbufA, nbufA, semLA, semNA)
        st = _chunk_scan(lbufA, nbufA, invts, st, c0)

        @pl.when(c0 + 2 < _NCH)
        def _():
            start(c0 + 2, lbufA, nbufA, semLA, semNA)

        wait(lbufB, nbufB, semLB, semNB)
        st = _chunk_scan(lbufB, nbufB, invts, st, c0 + 1)

        @pl.when(c0 + 3 < _NCH)
        def _():
            start(c0 + 3, lbufB, nbufB, semLB, semNB)

        return tuple(st)

    st = list(lax.fori_loop(0, _NCH // 2, pair, state0))

    tids = []
    for r in range(_TPG):
        best = jnp.max(st[2 * r])
        tids.append(jnp.min(jnp.where(st[2 * r] == best, st[2 * r + 1], _BIG)))

    def start2(tid, lbuf, nbuf, semL, semN):
        loff = pl.multiple_of(base + tid * 128, 128)
        noff = pl.multiple_of(base + lax.div(tid, 8) * 1024, 1024)
        pltpu.async_copy(
            logits.at[pl.ds(rows0, _TPG), pl.ds(loff, 128)],
            lbuf.at[:, pl.ds(0, 128)], semL)
        pltpu.async_copy(ntab.at[pl.ds(noff, 1024)],
                         nbuf.at[pl.ds(0, 1024)], semN)

    def wait2(lbuf, nbuf, semL, semN):
        pltpu.make_async_copy(
            logits.at[pl.ds(0, _TPG), pl.ds(0, 128)],
            lbuf.at[:, pl.ds(0, 128)], semL).wait()
        pltpu.make_async_copy(
            ntab.at[pl.ds(0, 1024)], nbuf.at[pl.ds(0, 1024)], semN).wait()

    start2(tids[0], lbufA, nbufA, semLA, semNA)
    start2(tids[1], lbufB, nbufB, semLB, semNB)
    for r in range(_TPG):
        lbuf, nbuf = (lbufA, nbufA) if r % 2 == 0 else (lbufB, nbufB)
        semL, semN = (semLA, semNA) if r % 2 == 0 else (semLB, semNB)
        wait2(lbuf, nbuf, semL, semN)
        nsub = lax.rem(tids[r], 8) * 128
        m, ix = _locate(lbuf, nbuf, nsub, invts[r], r,
                        base + tids[r] * 128, iota)
        if r + 2 < _TPG:
            start2(tids[r + 2], lbuf, nbuf, semL, semN)
        sval[r, :] = m
        sidx[r, :] = ix
    pltpu.sync_copy(sval, pval.at[wid])
    pltpu.sync_copy(sidx, pidx.at[wid])


_sampler = functools.partial(
    pl.kernel,
    out_type=(
        jax.ShapeDtypeStruct((_NC * _NS, _TPG, _L), jnp.float32),
        jax.ShapeDtypeStruct((_NC * _NS, _TPG, _L), jnp.int32),
    ),
    mesh=plsc.VectorSubcoreMesh(
        core_axis_name="c", subcore_axis_name="s",
        num_cores=_NC, num_subcores=_NS),
    compiler_params=pltpu.CompilerParams(needs_layout_passes=False),
    scratch_types=[
        pltpu.VMEM((_TPG, _CHUNK), jnp.float32),
        pltpu.VMEM((_TPG, _CHUNK), jnp.float32),
        pltpu.VMEM((_CHUNK,), jnp.float32),
        pltpu.VMEM((_CHUNK,), jnp.float32),
        pltpu.VMEM((_TOKENS,), jnp.float32),
        pltpu.VMEM((_TPG, _L), jnp.float32),
        pltpu.VMEM((_TPG, _L), jnp.int32),
        pltpu.SemaphoreType.DMA,
        pltpu.SemaphoreType.DMA,
        pltpu.SemaphoreType.DMA,
        pltpu.SemaphoreType.DMA,
    ],
)(_sampler_body)


def _suffix_body(l_ref, e_ref, invt_ref, ov_ref, oi_ref):
    b = pl.program_id(0)
    base = _TC_OFF + b * _TC_BW
    lane = jax.lax.broadcasted_iota(jnp.int32, (_TOKENS, _TC_BW), 1)
    nl = jnp.log(e_ref[0, :] + _EPS)
    s = l_ref[...] * invt_ref[...][:, None] - nl[None, :]
    s = jnp.where(base + lane < _VOCAB, s, -jnp.inf)
    ov_ref[...] = jnp.max(s, axis=1)[None, None, :]
    ai = jnp.argmax(s, axis=1).astype(jnp.int32) + base
    oi_ref[...] = ai[None, None, :]


def _suffix_scan(logits, exponential, invt):
    return pl.pallas_call(
        _suffix_body,
        grid=(_TC_NB,),
        in_specs=[
            pl.BlockSpec((_TOKENS, _TC_BW), lambda b: (0, _P + b)),
            pl.BlockSpec((1, _TC_BW), lambda b: (0, _P + b)),
            pl.BlockSpec((_TOKENS,), lambda b: (0,)),
        ],
        out_specs=[
            pl.BlockSpec((1, 1, _TOKENS), lambda b: (b, 0, 0)),
            pl.BlockSpec((1, 1, _TOKENS), lambda b: (b, 0, 0)),
        ],
        out_shape=[
            jax.ShapeDtypeStruct((_TC_NB, 1, _TOKENS), jnp.float32),
            jax.ShapeDtypeStruct((_TC_NB, 1, _TOKENS), jnp.int32),
        ],
    )(logits, exponential, invt)


def _merge_body(pv_ref, pi_ref, tv_ref, ti_ref, out_ref):
    v = pv_ref[...].reshape(_P, _G, _TPG, _L)
    ix = pi_ref[...].reshape(_P, _G, _TPG, _L)
    best = jnp.max(v, axis=(0, 3))
    tie = v == best[None, :, :, None]
    cand = jnp.where(tie, ix, _BIG)
    m = best.reshape(_TOKENS)
    i = jnp.min(cand, axis=(0, 3)).reshape(_TOKENS)

    tv = tv_ref[...]
    tmax = jnp.max(tv, axis=0)
    tcand = jnp.where(tv == tmax[None, :], ti_ref[...], _BIG)
    tidx = jnp.min(tcand, axis=0)
    out_ref[...] = jnp.where(tmax > m, tidx, i).astype(jnp.int32)


def _merge(pv, pi, tv, ti):
    nw = _NC * _NS
    return pl.pallas_call(
        _merge_body,
        grid=(1,),
        in_specs=[
            pl.BlockSpec((nw, _TPG, _L), lambda i: (0, 0, 0)),
            pl.BlockSpec((nw, _TPG, _L), lambda i: (0, 0, 0)),
            pl.BlockSpec((_TC_NB, _TOKENS), lambda i: (0, 0)),
            pl.BlockSpec((_TC_NB, _TOKENS), lambda i: (0, 0)),
        ],
        out_specs=pl.BlockSpec((_TOKENS,), lambda i: (0,)),
        out_shape=jax.ShapeDtypeStruct((_TOKENS,), jnp.int32),
    )(pv, pi, tv, ti)


@jax.jit
def kernel(logits, temperatures, exponential):
    ntab = _make_noise_table(exponential)
    pos = temperatures > 0
    invt = jnp.where(pos, 1.0 / jnp.where(pos, temperatures, 1.0), 1e30)
    pv, pi = _sampler(logits, invt, ntab)
    tv, ti = _suffix_scan(logits, exponential, invt)
    return _merge(pv, pi, tv.reshape(_TC_NB, _TOKENS),
                  ti.reshape(_TC_NB, _TOKENS))
